# Initial kernel scaffold; baseline (speedup 1.0000x reference)
import jax
import jax.numpy as jnp
from jax.experimental import pallas as pl


def kernel(x, edge_index, W1, b1, W2, b2, W3, b3):
    raise NotImplementedError("write your pallas kernel here")


# speedup vs baseline: 8.8954x; 8.8954x over previous
import functools

import jax
import jax.numpy as jnp
from jax import lax
from jax.experimental import pallas as pl
from jax.experimental.pallas import tpu as pltpu
from jax.experimental.pallas import tpu_sc as plsc

N = 10000
D = 128
NC, NS = 2, 16
NW = NC * NS
N_PAD = 10240
ROWS_PER_S = N_PAD // NS
CHUNK = 128
ROW_BLK = 1000
DEG_W = 16


@functools.cache
def _make_deg(e_pad: int):
    per_w = e_pad // NW
    n_chunks = per_w // CHUNK

    @functools.partial(
        pl.kernel,
        out_type=jax.ShapeDtypeStruct((NC, N_PAD, DEG_W), jnp.float32),
        mesh=plsc.VectorSubcoreMesh(
            core_axis_name="c", subcore_axis_name="s",
            num_cores=NC, num_subcores=NS),
        scratch_types=[
            pltpu.VMEM((CHUNK,), jnp.int32),
            pltpu.VMEM((CHUNK, DEG_W), jnp.float32),
            pltpu.VMEM_SHARED((N_PAD, DEG_W), jnp.float32),
        ])
    def deg(dst_hbm, ones_hbm, zeros_hbm, out_hbm, dst_v, ones_v, acc_sh):
        c = lax.axis_index("c")
        s = lax.axis_index("s")
        wid = s * NC + c
        r0 = pl.multiple_of(s * ROWS_PER_S, 8)
        pltpu.sync_copy(ones_hbm, ones_v)
        pltpu.sync_copy(zeros_hbm.at[pl.ds(r0, ROWS_PER_S)],
                        acc_sh.at[pl.ds(r0, ROWS_PER_S)])
        plsc.subcore_barrier()
        base = wid * per_w

        def body(i, carry):
            off = pl.multiple_of(base + i * CHUNK, 8)
            pltpu.sync_copy(dst_hbm.at[pl.ds(off, CHUNK)], dst_v)
            pltpu.sync_copy(ones_v, acc_sh.at[dst_v], add=True)
            return carry

        lax.fori_loop(0, n_chunks, body, 0)
        plsc.subcore_barrier()
        pltpu.sync_copy(acc_sh.at[pl.ds(r0, ROWS_PER_S)],
                        out_hbm.at[c, pl.ds(r0, ROWS_PER_S)])

    return deg


@functools.cache
def _make_agg(e_pad: int, width: int):
    per_w = e_pad // NW
    n_chunks = per_w // CHUNK

    @functools.partial(
        pl.kernel,
        out_type=jax.ShapeDtypeStruct((NC, N_PAD, width), jnp.float32),
        mesh=plsc.VectorSubcoreMesh(
            core_axis_name="c", subcore_axis_name="s",
            num_cores=NC, num_subcores=NS),
        scratch_types=[
            pltpu.VMEM((CHUNK,), jnp.int32),
            pltpu.VMEM((CHUNK,), jnp.int32),
            pltpu.VMEM((CHUNK, width), jnp.float32),
            pltpu.VMEM_SHARED((N_PAD, width), jnp.float32),
            pltpu.SemaphoreType.DMA,
        ])
    def agg(src_hbm, dst_hbm, table_hbm, zeros_hbm, out_hbm,
            src_v, dst_v, rows_v, acc_sh, sem):
        c = lax.axis_index("c")
        s = lax.axis_index("s")
        wid = s * NC + c
        r0 = pl.multiple_of(s * ROWS_PER_S, 8)
        pltpu.sync_copy(zeros_hbm.at[pl.ds(r0, ROWS_PER_S)],
                        acc_sh.at[pl.ds(r0, ROWS_PER_S)])
        plsc.subcore_barrier()
        base = wid * per_w

        def body(i, carry):
            off = pl.multiple_of(base + i * CHUNK, 8)
            pltpu.sync_copy(src_hbm.at[pl.ds(off, CHUNK)], src_v)
            pltpu.sync_copy(dst_hbm.at[pl.ds(off, CHUNK)], dst_v)
            pltpu.async_copy(table_hbm.at[src_v], rows_v, sem).wait()
            pltpu.sync_copy(rows_v, acc_sh.at[dst_v], add=True)
            return carry

        lax.fori_loop(0, n_chunks, body, 0)
        plsc.subcore_barrier()
        pltpu.sync_copy(acc_sh.at[pl.ds(r0, ROWS_PER_S)],
                        out_hbm.at[c, pl.ds(r0, ROWS_PER_S)])

    return agg


def _dis(deg):
    return lax.rsqrt(deg[0, :, :1] + deg[1, :, :1] + 1.0)


def _pre_body(deg_ref, x_ref, w_ref, o_ref):
    dis = _dis(deg_ref[...])
    o_ref[...] = dis * jnp.dot(x_ref[...], w_ref[...],
                               preferred_element_type=jnp.float32)


def _mid_body(deg_ref, parts_ref, yhat_ref, b_ref, w_ref, o_ref):
    dis = _dis(deg_ref[...])
    p = parts_ref[...]
    h = dis * (p[0] + p[1] + yhat_ref[...]) + b_ref[...]
    o_ref[...] = dis * jnp.dot(h, w_ref[...],
                               preferred_element_type=jnp.float32)


def _fin_body(deg_ref, parts_ref, yhat_ref, b_ref, o_ref):
    dis = _dis(deg_ref[...])
    p = parts_ref[...]
    o_ref[...] = dis * (p[0] + p[1] + yhat_ref[...]) + b_ref[...]


_DEG_SPEC = pl.BlockSpec((NC, ROW_BLK, DEG_W), lambda i: (0, i, 0))
_PARTS_SPEC = pl.BlockSpec((NC, ROW_BLK, D), lambda i: (0, i, 0))
_ROW_SPEC = pl.BlockSpec((ROW_BLK, D), lambda i: (i, 0))
_W_SPEC = pl.BlockSpec((D, D), lambda i: (0, 0))
_B_SPEC = pl.BlockSpec((1, D), lambda i: (0, 0))
_OUT = jax.ShapeDtypeStruct((N, D), jnp.float32)
_GRID = (N // ROW_BLK,)


def _tc_pre(deg_parts, x, w):
    return pl.pallas_call(
        _pre_body, grid=_GRID,
        in_specs=[_DEG_SPEC, _ROW_SPEC, _W_SPEC],
        out_specs=_ROW_SPEC, out_shape=_OUT,
    )(deg_parts, x, w)


def _tc_mid(deg_parts, parts, yhat, b, w):
    return pl.pallas_call(
        _mid_body, grid=_GRID,
        in_specs=[_DEG_SPEC, _PARTS_SPEC, _ROW_SPEC, _B_SPEC, _W_SPEC],
        out_specs=_ROW_SPEC, out_shape=_OUT,
    )(deg_parts, parts, yhat, b, w)


def _tc_fin(deg_parts, parts, yhat, b):
    return pl.pallas_call(
        _fin_body, grid=_GRID,
        in_specs=[_DEG_SPEC, _PARTS_SPEC, _ROW_SPEC, _B_SPEC],
        out_specs=_ROW_SPEC, out_shape=_OUT,
    )(deg_parts, parts, yhat, b)


def kernel(x, edge_index, W1, b1, W2, b2, W3, b3):
    src = edge_index[0].astype(jnp.int32)
    dst = edge_index[1].astype(jnp.int32)
    e = src.shape[0]
    e_pad = -(-e // (NW * CHUNK)) * (NW * CHUNK)
    pad = e_pad - e
    src_p = jnp.concatenate([src, jnp.zeros((pad,), jnp.int32)])
    dst_p = jnp.concatenate([dst, jnp.full((pad,), N, jnp.int32)])

    zeros_d = jnp.zeros((N_PAD, D), jnp.float32)
    zeros_g = jnp.zeros((N_PAD, DEG_W), jnp.float32)
    ones_g = jnp.ones((CHUNK, DEG_W), jnp.float32)

    agg_d = _make_agg(e_pad, D)

    deg_parts = _make_deg(e_pad)(dst_p, ones_g, zeros_g)
    b1r, b2r, b3r = (b.reshape(1, D) for b in (b1, b2, b3))

    yhat1 = _tc_pre(deg_parts, x, W1)
    parts1 = agg_d(src_p, dst_p, yhat1, zeros_d)
    yhat2 = _tc_mid(deg_parts, parts1, yhat1, b1r, W2)
    parts2 = agg_d(src_p, dst_p, yhat2, zeros_d)
    yhat3 = _tc_mid(deg_parts, parts2, yhat2, b2r, W3)
    parts3 = agg_d(src_p, dst_p, yhat3, zeros_d)
    return _tc_fin(deg_parts, parts3, yhat3, b3r)
